# Initial kernel scaffold; baseline (speedup 1.0000x reference)
import jax
import jax.numpy as jnp
from jax.experimental import pallas as pl


def kernel(x, W_in, W_rec):
    raise NotImplementedError("write your pallas kernel here")


# speedup vs baseline: 1.6021x; 1.6021x over previous
import jax
import jax.numpy as jnp
from jax.experimental import pallas as pl
from jax.experimental.pallas import tpu as pltpu

_N_INPUT = 28 * 28
_N_RES = 1000
_T = 25
_BETA = 0.95
_TH = 1.0

_K_PAD = 896
_N_PAD = 1024
_B_BLK = 512


def _lsm_body(x_ref, win_ref, wrec_ref, o_ref):
    x = x_ref[...]
    win = win_ref[...]
    wrec = wrec_ref[...]
    in_cur = jnp.dot(x, win, preferred_element_type=jnp.float32)

    mem = in_cur
    spk = (mem - _TH > 0).astype(jnp.float32)
    ssum = spk

    def step(_, carry):
        mem, spk, ssum = carry
        cur = in_cur + jnp.dot(spk, wrec, preferred_element_type=jnp.float32)
        reset = (mem - _TH > 0).astype(jnp.float32)
        mem = _BETA * mem + cur - reset * _TH
        spk = (mem - _TH > 0).astype(jnp.float32)
        return mem, spk, ssum + spk

    _, _, ssum = jax.lax.fori_loop(1, _T, step, (mem, spk, ssum))
    o_ref[...] = ssum * (1.0 / _T)


def kernel(x, W_in, W_rec):
    B = x.shape[0]
    x_p = jnp.pad(x, ((0, 0), (0, _K_PAD - _N_INPUT)))
    win_t = jnp.pad(W_in.T, ((0, _K_PAD - _N_INPUT), (0, _N_PAD - _N_RES)))
    wrec_t = jnp.pad(W_rec.T, ((0, _N_PAD - _N_RES), (0, _N_PAD - _N_RES)))

    out = pl.pallas_call(
        _lsm_body,
        out_shape=jax.ShapeDtypeStruct((B, _N_PAD), jnp.float32),
        grid=(B // _B_BLK,),
        in_specs=[
            pl.BlockSpec((_B_BLK, _K_PAD), lambda b: (b, 0)),
            pl.BlockSpec((_K_PAD, _N_PAD), lambda b: (0, 0)),
            pl.BlockSpec((_N_PAD, _N_PAD), lambda b: (0, 0)),
        ],
        out_specs=pl.BlockSpec((_B_BLK, _N_PAD), lambda b: (b, 0)),
        compiler_params=pltpu.CompilerParams(
            dimension_semantics=("parallel",),
        ),
        name="lsm_fused",
    )(x_p, win_t, wrec_t)
    return out[:, :_N_RES]


# speedup vs baseline: 1.7358x; 1.0835x over previous
import jax
import jax.numpy as jnp
from jax.experimental import pallas as pl
from jax.experimental.pallas import tpu as pltpu

_N_INPUT = 28 * 28
_N_RES = 1000
_T = 25
_BETA = 0.95
_TH = 1.0

_K_PAD = 896
_N_PAD = 1024
_B_BLK = 512


_B_HALF = _B_BLK // 2


def _lsm_body(x_ref, win_ref, wrec_ref, o_ref):
    win = win_ref[...]
    wrec = wrec_ref[...]
    icA = jnp.dot(x_ref[:_B_HALF], win, preferred_element_type=jnp.float32)
    icB = jnp.dot(x_ref[_B_HALF:], win, preferred_element_type=jnp.float32)

    memA = icA
    spkA = (memA - _TH > 0).astype(jnp.float32)
    ssA = spkA
    memB = icB
    spkB = (memB - _TH > 0).astype(jnp.float32)
    ssB = spkB

    def step(_, c):
        memA, spkA, ssA, memB, spkB, ssB = c
        recA = jnp.dot(spkA, wrec, preferred_element_type=jnp.float32)
        recB = jnp.dot(spkB, wrec, preferred_element_type=jnp.float32)
        memA = _BETA * memA + (icA + recA) - spkA * _TH
        memB = _BETA * memB + (icB + recB) - spkB * _TH
        spkA = (memA - _TH > 0).astype(jnp.float32)
        spkB = (memB - _TH > 0).astype(jnp.float32)
        return memA, spkA, ssA + spkA, memB, spkB, ssB + spkB

    _, _, ssA, _, _, ssB = jax.lax.fori_loop(
        1, _T, step, (memA, spkA, ssA, memB, spkB, ssB))
    o_ref[:_B_HALF] = ssA * (1.0 / _T)
    o_ref[_B_HALF:] = ssB * (1.0 / _T)


def kernel(x, W_in, W_rec):
    B = x.shape[0]
    x_p = jnp.pad(x, ((0, 0), (0, _K_PAD - _N_INPUT)))
    win_t = jnp.pad(W_in.T, ((0, _K_PAD - _N_INPUT), (0, _N_PAD - _N_RES)))
    wrec_t = jnp.pad(W_rec.T, ((0, _N_PAD - _N_RES), (0, _N_PAD - _N_RES)))

    out = pl.pallas_call(
        _lsm_body,
        out_shape=jax.ShapeDtypeStruct((B, _N_PAD), jnp.float32),
        grid=(B // _B_BLK,),
        in_specs=[
            pl.BlockSpec((_B_BLK, _K_PAD), lambda b: (b, 0)),
            pl.BlockSpec((_K_PAD, _N_PAD), lambda b: (0, 0)),
            pl.BlockSpec((_N_PAD, _N_PAD), lambda b: (0, 0)),
        ],
        out_specs=pl.BlockSpec((_B_BLK, _N_PAD), lambda b: (b, 0)),
        compiler_params=pltpu.CompilerParams(
            dimension_semantics=("parallel",),
        ),
        name="lsm_fused",
    )(x_p, win_t, wrec_t)
    return out[:, :_N_RES]
